# baseline (speedup 1.0000x reference)
import functools

import jax
import jax.numpy as jnp
from jax import lax
from jax.experimental import pallas as pl
from jax.experimental.pallas import tpu as pltpu
from jax.experimental.pallas import tpu_sc as plsc

N = 32768
H = 256
B = 16
BLK = 8192
CH = 8192
NB = N // BLK
WIN = N // 16


@functools.partial(
    pl.kernel,
    out_type=jax.ShapeDtypeStruct((B, H), jnp.float32),
    scratch_types=[
        pltpu.VMEM((16 * 8 + 16,), jnp.int32),
        pltpu.VMEM((WIN + 16,), jnp.int32),
        pltpu.VMEM((1, H), jnp.float32),
        pltpu.SemaphoreType.DMA,
    ],
    mesh=plsc.VectorSubcoreMesh(core_axis_name="c", subcore_axis_name="s"),
)
def _sc_gather_last(x_hbm, batch_hbm, vn_hbm, samp_v, win_v, row_v, sem):
    c = lax.axis_index("c")
    s = lax.axis_index("s")

    @pl.when(c == 0)
    def _():
        b = s
        cps = []
        for k in range(16):
            cp = pltpu.make_async_copy(
                batch_hbm.at[pl.ds((k + 1) * WIN - 8, 8)],
                samp_v.at[pl.ds(k * 8, 8)],
                sem)
            cp.start()
            cps.append(cp)
        for cp in cps:
            cp.wait()
        n_full = jnp.int32(0)
        for k in range(16):
            vk = samp_v[pl.ds(k * 8, 16)]
            n_full += jnp.where(vk[7] <= b, jnp.int32(1), jnp.int32(0))
        base = jnp.minimum(n_full, jnp.int32(15)) * WIN
        pltpu.sync_copy(batch_hbm.at[pl.ds(base, WIN)],
                        win_v.at[pl.ds(0, WIN)])
        pos = jnp.int32(0)
        sh = WIN // 2
        while sh >= 1:
            cand = pos + sh
            vec = win_v[pl.ds(cand - 1, 16)]
            le = vec[0] <= b
            pos = jnp.where(le, jnp.int32(cand), pos)
            sh //= 2
        last = jnp.where(b == B - 1, jnp.int32(N - 1), base + pos - 1)
        pltpu.sync_copy(x_hbm.at[pl.ds(last, 1), :], row_v)
        pltpu.sync_copy(row_v, vn_hbm.at[pl.ds(b, 1), :])


def _fused_kernel(x_ref,
                  batch_ref,
                  nc_ref,
                  w1_ref,
                  w2_ref,
                  b1_ref,
                  b2_ref,
                  q_ref,
                  qb_ref,
                  w3_ref,
                  b3_ref,
                  vn_ref,
                  out_ref,
                  a_ref,
                  sg_ref,
                  w2bf_ref):
    i = pl.program_id(0)

    @pl.when(i == 0)
    def _prologue():
        sg_ref[...] = jnp.zeros_like(sg_ref)
        a_ref[...] = (
            lax.dot_general(vn_ref[...], w1_ref[...],
                            (((1,), (1,)), ((), ())),
                            preferred_element_type=jnp.float32)
            + b1_ref[...] + b2_ref[...])
        w2bf_ref[...] = w2_ref[...].astype(jnp.bfloat16)

    b_iota = lax.broadcasted_iota(jnp.int32, (B, CH), 0)
    acc = None
    for c in range(BLK // CH):
        sl = pl.ds(c * CH, CH)
        x_bf = x_ref[sl, :].astype(jnp.bfloat16)
        batch_row = batch_ref[0, :, sl]
        oh_t = (batch_row == b_iota).astype(jnp.float32)
        m = lax.dot_general(x_bf, w2bf_ref[...],
                            (((1,), (1,)), ((), ())),
                            preferred_element_type=jnp.float32)
        a_tok = lax.dot_general(oh_t, a_ref[...],
                                (((0,), (0,)), ((), ())),
                                preferred_element_type=jnp.float32)
        z = m + a_tok
        gate = 1.0 / (1.0 + jnp.exp2(z * (-1.4426950408889634)))
        alpha = lax.dot_general(q_ref[...], gate,
                                (((1,), (1,)), ((), ())),
                                preferred_element_type=jnp.float32)
        scale = (alpha + qb_ref[...]) * nc_ref[0, :, sl]
        d = jnp.dot((oh_t * scale).astype(jnp.bfloat16), x_bf,
                    preferred_element_type=jnp.float32)
        acc = d if acc is None else acc + d
    sg_ref[...] += acc

    @pl.when(i == NB - 1)
    def _epilogue():
        out_ref[...] = (
            lax.dot_general(vn_ref[...], w3_ref[:, :H],
                            (((1,), (1,)), ((), ())),
                            preferred_element_type=jnp.float32)
            + lax.dot_general(sg_ref[...], w3_ref[:, H:],
                              (((1,), (1,)), ((), ())),
                              preferred_element_type=jnp.float32)
            + b3_ref[...])


@jax.jit
def kernel(node_embedding, batch, num_count, W1_w, W1_b, W2_w, W2_b,
           q_w, q_b, W3_w, W3_b):
    vn = _sc_gather_last(node_embedding, batch)

    batch_blk = batch.reshape(NB, 1, BLK)
    nc_blk = num_count.reshape(NB, 1, BLK)

    const2 = lambda i: (0, 0)
    out = pl.pallas_call(
        _fused_kernel,
        grid=(NB,),
        in_specs=[
            pl.BlockSpec((BLK, H), lambda i: (i, 0)),
            pl.BlockSpec((1, 1, BLK), lambda i: (i, 0, 0)),
            pl.BlockSpec((1, 1, BLK), lambda i: (i, 0, 0)),
            pl.BlockSpec((H, H), const2),
            pl.BlockSpec((H, H), const2),
            pl.BlockSpec((1, H), const2),
            pl.BlockSpec((1, H), const2),
            pl.BlockSpec((1, H), const2),
            pl.BlockSpec((1, 1), const2),
            pl.BlockSpec((H, 2 * H), const2),
            pl.BlockSpec((1, H), const2),
            pl.BlockSpec((B, H), const2),
        ],
        out_specs=pl.BlockSpec((B, H), const2),
        out_shape=jax.ShapeDtypeStruct((B, H), jnp.float32),
        scratch_shapes=[
            pltpu.VMEM((B, H), jnp.float32),
            pltpu.VMEM((B, H), jnp.float32),
            pltpu.VMEM((H, H), jnp.bfloat16),
        ],
        compiler_params=pltpu.CompilerParams(
            dimension_semantics=("arbitrary",)),
    )(node_embedding, batch_blk, nc_blk,
      W1_w, W2_w, W1_b.reshape(1, H), W2_b.reshape(1, H), q_w,
      q_b.reshape(1, 1), W3_w, W3_b.reshape(1, H), vn)
    return out


# speedup vs baseline: 1.0117x; 1.0117x over previous
import functools

import jax
import jax.numpy as jnp
from jax import lax
from jax.experimental import pallas as pl
from jax.experimental.pallas import tpu as pltpu
from jax.experimental.pallas import tpu_sc as plsc

N = 32768
H = 256
B = 16
BLK = 4096
CH = 4096
NB = N // BLK
WIN = N // 16


@functools.partial(
    pl.kernel,
    out_type=jax.ShapeDtypeStruct((B, H), jnp.float32),
    scratch_types=[
        pltpu.VMEM((16 * 8 + 16,), jnp.int32),
        pltpu.VMEM((WIN + 16,), jnp.int32),
        pltpu.VMEM((1, H), jnp.float32),
        pltpu.SemaphoreType.DMA,
    ],
    mesh=plsc.VectorSubcoreMesh(core_axis_name="c", subcore_axis_name="s"),
)
def _sc_gather_last(x_hbm, batch_hbm, vn_hbm, samp_v, win_v, row_v, sem):
    c = lax.axis_index("c")
    s = lax.axis_index("s")

    @pl.when(c == 0)
    def _():
        b = s
        cps = []
        for k in range(16):
            cp = pltpu.make_async_copy(
                batch_hbm.at[pl.ds((k + 1) * WIN - 8, 8)],
                samp_v.at[pl.ds(k * 8, 8)],
                sem)
            cp.start()
            cps.append(cp)
        for cp in cps:
            cp.wait()
        n_full = jnp.int32(0)
        for k in range(16):
            vk = samp_v[pl.ds(k * 8, 16)]
            n_full += jnp.where(vk[7] <= b, jnp.int32(1), jnp.int32(0))
        base = jnp.minimum(n_full, jnp.int32(15)) * WIN
        pltpu.sync_copy(batch_hbm.at[pl.ds(base, WIN)],
                        win_v.at[pl.ds(0, WIN)])
        pos = jnp.int32(0)
        sh = WIN // 2
        while sh >= 1:
            cand = pos + sh
            vec = win_v[pl.ds(cand - 1, 16)]
            le = vec[0] <= b
            pos = jnp.where(le, jnp.int32(cand), pos)
            sh //= 2
        last = jnp.where(b == B - 1, jnp.int32(N - 1), base + pos - 1)
        pltpu.sync_copy(x_hbm.at[pl.ds(last, 1), :], row_v)
        pltpu.sync_copy(row_v, vn_hbm.at[pl.ds(b, 1), :])


def _fused_kernel(x_ref,
                  batch_ref,
                  nc_ref,
                  w1_ref,
                  w2_ref,
                  b1_ref,
                  b2_ref,
                  q_ref,
                  qb_ref,
                  w3_ref,
                  b3_ref,
                  vn_ref,
                  out_ref,
                  a_ref,
                  sg_ref,
                  w2bf_ref):
    i = pl.program_id(0)

    @pl.when(i == 0)
    def _prologue():
        sg_ref[...] = jnp.zeros_like(sg_ref)
        a_ref[...] = (
            lax.dot_general(vn_ref[...], w1_ref[...],
                            (((1,), (1,)), ((), ())),
                            preferred_element_type=jnp.float32)
            + b1_ref[...] + b2_ref[...])
        w2bf_ref[...] = w2_ref[...].astype(jnp.bfloat16)

    b_iota = lax.broadcasted_iota(jnp.int32, (B, CH), 0)
    acc = None
    for c in range(BLK // CH):
        sl = pl.ds(c * CH, CH)
        x_bf = x_ref[sl, :].astype(jnp.bfloat16)
        batch_row = batch_ref[0, :, sl]
        oh_t = (batch_row == b_iota).astype(jnp.float32)
        m = lax.dot_general(x_bf, w2bf_ref[...],
                            (((1,), (1,)), ((), ())),
                            preferred_element_type=jnp.float32)
        a_tok = lax.dot_general(oh_t, a_ref[...],
                                (((0,), (0,)), ((), ())),
                                preferred_element_type=jnp.float32)
        z = m + a_tok
        gate = 1.0 / (1.0 + jnp.exp2(z * (-1.4426950408889634)))
        alpha = lax.dot_general(q_ref[...], gate,
                                (((1,), (1,)), ((), ())),
                                preferred_element_type=jnp.float32)
        scale = (alpha + qb_ref[...]) * nc_ref[0, :, sl]
        d = jnp.dot((oh_t * scale).astype(jnp.bfloat16), x_bf,
                    preferred_element_type=jnp.float32)
        acc = d if acc is None else acc + d
    sg_ref[...] += acc

    @pl.when(i == NB - 1)
    def _epilogue():
        out_ref[...] = (
            lax.dot_general(vn_ref[...], w3_ref[:, :H],
                            (((1,), (1,)), ((), ())),
                            preferred_element_type=jnp.float32)
            + lax.dot_general(sg_ref[...], w3_ref[:, H:],
                              (((1,), (1,)), ((), ())),
                              preferred_element_type=jnp.float32)
            + b3_ref[...])


@jax.jit
def kernel(node_embedding, batch, num_count, W1_w, W1_b, W2_w, W2_b,
           q_w, q_b, W3_w, W3_b):
    vn = _sc_gather_last(node_embedding, batch)

    batch_blk = batch.reshape(NB, 1, BLK)
    nc_blk = num_count.reshape(NB, 1, BLK)

    const2 = lambda i: (0, 0)
    out = pl.pallas_call(
        _fused_kernel,
        grid=(NB,),
        in_specs=[
            pl.BlockSpec((BLK, H), lambda i: (i, 0)),
            pl.BlockSpec((1, 1, BLK), lambda i: (i, 0, 0)),
            pl.BlockSpec((1, 1, BLK), lambda i: (i, 0, 0)),
            pl.BlockSpec((H, H), const2),
            pl.BlockSpec((H, H), const2),
            pl.BlockSpec((1, H), const2),
            pl.BlockSpec((1, H), const2),
            pl.BlockSpec((1, H), const2),
            pl.BlockSpec((1, 1), const2),
            pl.BlockSpec((H, 2 * H), const2),
            pl.BlockSpec((1, H), const2),
            pl.BlockSpec((B, H), const2),
        ],
        out_specs=pl.BlockSpec((B, H), const2),
        out_shape=jax.ShapeDtypeStruct((B, H), jnp.float32),
        scratch_shapes=[
            pltpu.VMEM((B, H), jnp.float32),
            pltpu.VMEM((B, H), jnp.float32),
            pltpu.VMEM((H, H), jnp.bfloat16),
        ],
        compiler_params=pltpu.CompilerParams(
            dimension_semantics=("arbitrary",)),
    )(node_embedding, batch_blk, nc_blk,
      W1_w, W2_w, W1_b.reshape(1, H), W2_b.reshape(1, H), q_w,
      q_b.reshape(1, 1), W3_w, W3_b.reshape(1, H), vn)
    return out
